# baseline (speedup 1.0000x reference)
import functools

import jax
import jax.numpy as jnp
from jax import lax
from jax.experimental import pallas as pl
from jax.experimental.pallas import tpu as pltpu
from jax.experimental.pallas import tpu_sc as plsc

_LN2 = 0.6931471805599453


def kernel(x, neutral_idx, W):
    B = x.shape[0]
    V = W.shape[0]
    Wf = W.reshape(V)

    info = plsc.get_sparse_core_info()
    NC, NS, L = info.num_cores, info.num_subcores, info.num_lanes
    NW = NC * NS
    b_per_w = B // NW
    CH = 128
    K = b_per_w // CH

    x_r = x.reshape(NW, K, CH)
    n_idx = jnp.full((L,), neutral_idx, dtype=jnp.int32)
    mesh = plsc.VectorSubcoreMesh(core_axis_name="c", subcore_axis_name="s")

    @functools.partial(
        pl.kernel,
        mesh=mesh,
        out_type=jax.ShapeDtypeStruct((NW, K, CH), jnp.float32),
        scratch_types=[
            pltpu.VMEM((K, CH), jnp.int32),
            pltpu.VMEM((K, CH), jnp.float32),
            pltpu.VMEM((L,), jnp.int32),
            pltpu.VMEM((L,), jnp.float32),
            pltpu.VMEM((K, CH), jnp.float32),
            pltpu.SemaphoreType.DMA((K,)),
            pltpu.SemaphoreType.DMA((K,)),
            pltpu.SemaphoreType.DMA,
        ],
    )
    def run(table_hbm, nidx_hbm, xr_hbm, out_hbm,
            idx_v, vals_v, nidx_v, nval_v, out_v, ssem, gsem, nsem):
        wid = lax.axis_index("s") * NC + lax.axis_index("c")
        stages = [
            pltpu.async_copy(xr_hbm.at[wid].at[j], idx_v.at[j], ssem.at[j])
            for j in range(K)
        ]
        gathers = []
        stages[0].wait()
        gathers.append(
            pltpu.async_copy(table_hbm.at[idx_v.at[0]], vals_v.at[0],
                             gsem.at[0]))
        pltpu.sync_copy(nidx_hbm, nidx_v)
        nc = pltpu.async_copy(table_hbm.at[nidx_v], nval_v, nsem)
        for j in range(1, K):
            stages[j].wait()
            gathers.append(
                pltpu.async_copy(table_hbm.at[idx_v.at[j]], vals_v.at[j],
                                 gsem.at[j]))
        nc.wait()
        nvec = nval_v[...]
        for j in range(K):
            gathers[j].wait()
            for i in range(CH // L):
                v = vals_v[j, pl.ds(i * L, L)]
                out_v[j, pl.ds(i * L, L)] = jnp.exp((v - nvec) * _LN2)
        pltpu.sync_copy(out_v, out_hbm.at[wid])

    out = run(Wf, n_idx, x_r)
    return out.reshape(B, 1)


# speedup vs baseline: 1.0032x; 1.0032x over previous
import functools

import jax
import jax.numpy as jnp
from jax import lax
from jax.experimental import pallas as pl
from jax.experimental.pallas import tpu as pltpu
from jax.experimental.pallas import tpu_sc as plsc

_LN2 = 0.6931471805599453


def kernel(x, neutral_idx, W):
    B = x.shape[0]
    V = W.shape[0]
    Wf = W.reshape(V)

    info = plsc.get_sparse_core_info()
    NC, NS, L = info.num_cores, info.num_subcores, info.num_lanes
    NW = NC * NS
    b_per_w = B // NW
    CH = 128
    K = b_per_w // CH

    x_r = x.reshape(NW, K, CH)
    n_idx = jnp.full((L,), neutral_idx, dtype=jnp.int32)
    mesh = plsc.VectorSubcoreMesh(core_axis_name="c", subcore_axis_name="s")

    @functools.partial(
        pl.kernel,
        mesh=mesh,
        out_type=jax.ShapeDtypeStruct((NW, K, CH), jnp.float32),
        scratch_types=[
            pltpu.VMEM((K, CH), jnp.int32),
            pltpu.VMEM((K, CH), jnp.float32),
            pltpu.VMEM((L,), jnp.int32),
            pltpu.VMEM((L,), jnp.float32),
            pltpu.VMEM((K, CH), jnp.float32),
            pltpu.SemaphoreType.DMA((K,)),
            pltpu.SemaphoreType.DMA((K,)),
            pltpu.SemaphoreType.DMA,
            pltpu.SemaphoreType.DMA,
        ],
    )
    def run(table_hbm, nidx_hbm, xr_hbm, out_hbm,
            idx_v, vals_v, nidx_v, nval_v, out_v, ssem, gsem, nsem, wsem):
        wid = lax.axis_index("s") * NC + lax.axis_index("c")
        stages = [
            pltpu.async_copy(xr_hbm.at[wid].at[j], idx_v.at[j], ssem.at[j])
            for j in range(K)
        ]
        pltpu.sync_copy(nidx_hbm, nidx_v)
        nc = pltpu.async_copy(table_hbm.at[nidx_v], nval_v, nsem)
        gathers = []
        for j in range(K):
            stages[j].wait()
            gathers.append(
                pltpu.async_copy(table_hbm.at[idx_v.at[j]], vals_v.at[j],
                                 gsem.at[j]))
        nc.wait()
        nvec = nval_v[...]
        writes = []
        for j in range(K):
            gathers[j].wait()
            for i in range(CH // L):
                v = vals_v[j, pl.ds(i * L, L)]
                out_v[j, pl.ds(i * L, L)] = jnp.exp((v - nvec) * _LN2)
            writes.append(
                pltpu.async_copy(out_v.at[j], out_hbm.at[wid].at[j], wsem))
        for w in writes:
            w.wait()

    out = run(Wf, n_idx, x_r)
    return out.reshape(B, 1)


# speedup vs baseline: 1.0314x; 1.0281x over previous
import functools

import jax
import jax.numpy as jnp
from jax import lax
from jax.experimental import pallas as pl
from jax.experimental.pallas import tpu as pltpu
from jax.experimental.pallas import tpu_sc as plsc

_LN2 = 0.6931471805599453


def kernel(x, neutral_idx, W):
    B = x.shape[0]
    V = W.shape[0]
    Wf = W.reshape(V)

    info = plsc.get_sparse_core_info()
    NC, NS, L = info.num_cores, info.num_subcores, info.num_lanes
    NC = 1
    NW = NC * NS
    b_per_w = B // NW
    CH = 128
    K = b_per_w // CH

    x_r = x.reshape(NW, K, CH)
    n_idx = jnp.full((L,), neutral_idx, dtype=jnp.int32)
    mesh = plsc.VectorSubcoreMesh(core_axis_name="c", subcore_axis_name="s",
                                  num_cores=NC)

    @functools.partial(
        pl.kernel,
        mesh=mesh,
        out_type=jax.ShapeDtypeStruct((NW, K, CH), jnp.float32),
        scratch_types=[
            pltpu.VMEM((K, CH), jnp.int32),
            pltpu.VMEM((K, CH), jnp.float32),
            pltpu.VMEM((L,), jnp.int32),
            pltpu.VMEM((L,), jnp.float32),
            pltpu.VMEM((K, CH), jnp.float32),
            pltpu.SemaphoreType.DMA((K,)),
            pltpu.SemaphoreType.DMA((K,)),
            pltpu.SemaphoreType.DMA,
            pltpu.SemaphoreType.DMA,
        ],
    )
    def run(table_hbm, nidx_hbm, xr_hbm, out_hbm,
            idx_v, vals_v, nidx_v, nval_v, out_v, ssem, gsem, nsem, wsem):
        wid = lax.axis_index("s") * NC + lax.axis_index("c")
        stages = [
            pltpu.async_copy(xr_hbm.at[wid].at[j], idx_v.at[j], ssem.at[j])
            for j in range(K)
        ]
        pltpu.sync_copy(nidx_hbm, nidx_v)
        nc = pltpu.async_copy(table_hbm.at[nidx_v], nval_v, nsem)
        gathers = []
        for j in range(K):
            stages[j].wait()
            gathers.append(
                pltpu.async_copy(table_hbm.at[idx_v.at[j]], vals_v.at[j],
                                 gsem.at[j]))
        nc.wait()
        nvec = nval_v[...]
        writes = []
        for j in range(K):
            gathers[j].wait()
            for i in range(CH // L):
                v = vals_v[j, pl.ds(i * L, L)]
                out_v[j, pl.ds(i * L, L)] = jnp.exp((v - nvec) * _LN2)
            writes.append(
                pltpu.async_copy(out_v.at[j], out_hbm.at[wid].at[j], wsem))
        for w in writes:
            w.wait()

    out = run(Wf, n_idx, x_r)
    return out.reshape(B, 1)


# speedup vs baseline: 1.0321x; 1.0007x over previous
import functools

import jax
import jax.numpy as jnp
from jax import lax
from jax.experimental import pallas as pl
from jax.experimental.pallas import tpu as pltpu
from jax.experimental.pallas import tpu_sc as plsc

_LN2 = 0.6931471805599453


def kernel(x, neutral_idx, W):
    B = x.shape[0]
    V = W.shape[0]
    Wf = W.reshape(V)

    info = plsc.get_sparse_core_info()
    NC, NS, L = info.num_cores, info.num_subcores, info.num_lanes
    NC = 1
    NW = NC * NS
    b_per_w = B // NW
    CH = 128
    K = b_per_w // CH

    x_r = x.reshape(NW, K, CH)
    n_idx = jnp.full((L,), neutral_idx, dtype=jnp.int32)
    mesh = plsc.VectorSubcoreMesh(core_axis_name="c", subcore_axis_name="s",
                                  num_cores=NC)

    @functools.partial(
        pl.kernel,
        mesh=mesh,
        compiler_params=pltpu.CompilerParams(
            disable_bounds_checks=True,
            disable_semaphore_checks=True,
        ),
        out_type=jax.ShapeDtypeStruct((NW, K, CH), jnp.float32),
        scratch_types=[
            pltpu.VMEM((K, CH), jnp.int32),
            pltpu.VMEM((K, CH), jnp.float32),
            pltpu.VMEM((L,), jnp.int32),
            pltpu.VMEM((L,), jnp.float32),
            pltpu.VMEM((K, CH), jnp.float32),
            pltpu.SemaphoreType.DMA((K,)),
            pltpu.SemaphoreType.DMA((K,)),
            pltpu.SemaphoreType.DMA,
            pltpu.SemaphoreType.DMA,
        ],
    )
    def run(table_hbm, nidx_hbm, xr_hbm, out_hbm,
            idx_v, vals_v, nidx_v, nval_v, out_v, ssem, gsem, nsem, wsem):
        wid = lax.axis_index("s") * NC + lax.axis_index("c")
        stages = [
            pltpu.async_copy(xr_hbm.at[wid].at[j], idx_v.at[j], ssem.at[j])
            for j in range(K)
        ]
        pltpu.sync_copy(nidx_hbm, nidx_v)
        nc = pltpu.async_copy(table_hbm.at[nidx_v], nval_v, nsem)
        gathers = []
        for j in range(K):
            stages[j].wait()
            gathers.append(
                pltpu.async_copy(table_hbm.at[idx_v.at[j]], vals_v.at[j],
                                 gsem.at[j]))
        nc.wait()
        nvec = nval_v[...]
        writes = []
        for j in range(K):
            gathers[j].wait()
            for i in range(CH // L):
                v = vals_v[j, pl.ds(i * L, L)]
                out_v[j, pl.ds(i * L, L)] = jnp.exp((v - nvec) * _LN2)
            writes.append(
                pltpu.async_copy(out_v.at[j], out_hbm.at[wid].at[j], wsem))
        for w in writes:
            w.wait()

    out = run(Wf, n_idx, x_r)
    return out.reshape(B, 1)


# speedup vs baseline: 1.0358x; 1.0036x over previous
import functools

import jax
import jax.numpy as jnp
from jax import lax
from jax.experimental import pallas as pl
from jax.experimental.pallas import tpu as pltpu
from jax.experimental.pallas import tpu_sc as plsc

_LN2 = 0.6931471805599453


def kernel(x, neutral_idx, W):
    B = x.shape[0]
    V = W.shape[0]
    Wf = W.reshape(V)

    info = plsc.get_sparse_core_info()
    NC, NS, L = info.num_cores, info.num_subcores, info.num_lanes
    NC = 1
    NW = NC * NS
    b_per_w = B // NW
    CH = 128
    K = b_per_w // CH

    x_r = x.reshape(NW, K, CH)
    n_idx = jnp.full((L,), neutral_idx, dtype=jnp.int32)
    mesh = plsc.VectorSubcoreMesh(core_axis_name="c", subcore_axis_name="s",
                                  num_cores=NC)

    @functools.partial(
        pl.kernel,
        mesh=mesh,
        out_type=jax.ShapeDtypeStruct((NW, K, CH), jnp.float32),
        scratch_types=[
            pltpu.VMEM((K, CH), jnp.int32),
            pltpu.VMEM((K, CH), jnp.float32),
            pltpu.VMEM((L,), jnp.int32),
            pltpu.VMEM((L,), jnp.float32),
            pltpu.VMEM((K, CH), jnp.float32),
            pltpu.SemaphoreType.DMA((K,)),
            pltpu.SemaphoreType.DMA((K,)),
            pltpu.SemaphoreType.DMA,
            pltpu.SemaphoreType.DMA,
        ],
    )
    def run(table_hbm, nidx_hbm, xr_hbm, out_hbm,
            idx_v, vals_v, nidx_v, nval_v, out_v, ssem, gsem, nsem, wsem):
        wid = lax.axis_index("s") * NC + lax.axis_index("c")
        stage = pltpu.async_copy(xr_hbm.at[wid], idx_v, ssem.at[0])
        pltpu.sync_copy(nidx_hbm, nidx_v)
        nc = pltpu.async_copy(table_hbm.at[nidx_v], nval_v, nsem)
        stage.wait()
        gathers = [
            pltpu.async_copy(table_hbm.at[idx_v.at[j]], vals_v.at[j],
                             gsem.at[j])
            for j in range(K)
        ]
        nc.wait()
        nvec = nval_v[...]
        writes = []
        for j in range(K):
            gathers[j].wait()
            for i in range(CH // L):
                v = vals_v[j, pl.ds(i * L, L)]
                out_v[j, pl.ds(i * L, L)] = jnp.exp((v - nvec) * _LN2)
            writes.append(
                pltpu.async_copy(out_v.at[j], out_hbm.at[wid].at[j], wsem))
        for w in writes:
            w.wait()

    out = run(Wf, n_idx, x_r)
    return out.reshape(B, 1)
